# baseline (speedup 1.0000x reference)
import jax
import jax.numpy as jnp
import numpy as np
from jax.experimental import pallas as pl
from jax.experimental.pallas import tpu as pltpu

BB = 16
PER_STEP = 2
KOBJ = 100
EMB = 1024
FEAT = 2052
HID = 1024
OUTD = 3000
NK = 8
NS = 16
COMB = 512

_TWO_PI = np.float32(2.0 * np.pi)
_NEG = np.float32(-3.0e38)


def _f32dot(a, b):
    return jnp.dot(a, b, preferred_element_type=jnp.float32)


def _ntdot(a, b):
    return jax.lax.dot_general(a, b, (((1,), (1,)), ((), ())),
                               preferred_element_type=jnp.float32)


def _top16_mask(adj):
    cur = adj
    c = jnp.zeros((1, KOBJ), jnp.float32)
    v16 = jnp.zeros((1, KOBJ), jnp.float32)
    g = jnp.zeros((1, KOBJ), jnp.float32)
    rowmax = jnp.max(adj, axis=0, keepdims=True)
    for _ in range(NS):
        t = jnp.max(cur, axis=0, keepdims=True)
        hit = cur >= t
        nh = jnp.sum(hit.astype(jnp.float32), axis=0, keepdims=True)
        active = c < float(NS)
        v16 = jnp.where(active, t, v16)
        g = jnp.where(active, c, g)
        c = c + nh
        cur = jnp.where(hit, _NEG, cur)
    v16c = v16.T
    gc = g.T
    gt = adj > v16c
    eq = adj == v16c
    lidx = jax.lax.broadcasted_iota(jnp.int32, (KOBJ, KOBJ), 0)
    jidx = jax.lax.broadcasted_iota(jnp.int32, (KOBJ, KOBJ), 1)
    upper = (lidx < jidx).astype(jnp.float32)
    prefix = _f32dot(eq.astype(jnp.float32), upper)
    sel = jnp.logical_or(gt, jnp.logical_and(eq, prefix < (float(NS) - gc)))
    return sel, rowmax.T


def _kernel_weights(p_ref, rho, theta):
    ws = []
    wsum = jnp.zeros((KOBJ, KOBJ), jnp.float32)
    for m in range(NK):
        mr = p_ref[0, m]
        mt = p_ref[1, m]
        pr = p_ref[2, m]
        pt = p_ref[3, m]
        cr = -0.5 / (1e-14 + pr * pr)
        ct = -0.5 / (1e-14 + pt * pt)
        d = rho - mr
        wr = jnp.exp(d * d * cr)
        fa = jnp.abs(theta - mt)
        sa = jnp.abs(_TWO_PI - fa)
        mn = jnp.minimum(fa, sa)
        wt = jnp.exp(mn * mn * ct)
        w = wr * wt
        ws.append(w)
        wsum = wsum + w
    return ws, wsum


def _fused_body(qall_ref, img_ref, wlp_ref, blp_ref, w1_ref, b1_ref, w2_ref,
                b2_ref, p1_ref, p2_ref, wg1_ref, wg2_ref, adj_ref, hq_ref,
                qenc_s, qp_s, wg1_s, wg2_s):
    b = pl.program_id(0)

    @pl.when(b == 0)
    def _init():
        qe = _ntdot(qall_ref[...], wlp_ref[...]) + blp_ref[...]
        qenc_s[...] = qe
        qp_s[...] = _ntdot(qe, w1_ref[:, FEAT:])
        wg1_s[...] = wg1_ref[...].astype(jnp.bfloat16)
        wg2_s[...] = wg2_ref[...].astype(jnp.bfloat16)

    for e in range(PER_STEP):
        _one_example(b * PER_STEP + e, e, img_ref, w1_ref, b1_ref, w2_ref,
                     b2_ref, p1_ref, p2_ref, adj_ref, hq_ref,
                     qenc_s, qp_s, wg1_s, wg2_s)


def _one_example(bex, e, img_ref, w1_ref, b1_ref, w2_ref, b2_ref,
                 p1_ref, p2_ref, adj_ref, hq_ref, qenc_s, qp_s, wg1_s, wg2_s):
    qenc = qenc_s[pl.ds(bex, 1), :]
    qpart = qp_s[pl.ds(bex, 1), :]
    img = img_ref[e]
    h = jnp.maximum(
        _ntdot(img, w1_ref[:, :FEAT]) + qpart + b1_ref[...], 0.0)
    h = jnp.maximum(_ntdot(h, w2_ref[...]) + b2_ref[...], 0.0)
    adj = _ntdot(h, h)
    adj_ref[e] = adj

    bb = img[:, FEAT - 4:]
    cx = 0.5 * (bb[:, 0:1] + bb[:, 2:3])
    cy = 0.5 * (bb[:, 1:2] + bb[:, 3:4])
    dx = cx - cx.T
    dy = cy - cy.T
    rho = jnp.sqrt(dx * dx + dy * dy)
    theta = jnp.arctan2(dx, dy)

    sel, rowmaxc = _top16_mask(adj)
    ex = jnp.where(sel, jnp.exp(adj - rowmaxc), 0.0)
    sm = ex * (1.0 / jnp.sum(ex, axis=1, keepdims=True))
    self32 = sel.astype(jnp.float32)
    bf16 = jnp.bfloat16
    imgb = img.astype(bf16)

    ws1, wsum1 = _kernel_weights(p1_ref, rho, theta)
    s1 = sm * (1.0 / wsum1)
    h1parts = []
    for m in range(NK):
        y = _f32dot(imgb, wg1_s[m])
        a1m = jnp.where(sel, ws1[m] * s1, 0.0)
        h1parts.append(
            jnp.maximum(_f32dot(a1m.astype(bf16), y.astype(bf16)), 0.0))
    h1 = jnp.concatenate(h1parts, axis=1)

    ws2, wsum2 = _kernel_weights(p2_ref, rho, theta)
    s2 = self32 * (1.0 / wsum2)
    h1b = h1.astype(bf16)
    h2parts = []
    for m in range(NK):
        y2 = _f32dot(h1b, wg2_s[m])
        a2m = jnp.where(sel, ws2[m] * s2, 0.0)
        om = jnp.maximum(_f32dot(a2m.astype(bf16), y2.astype(bf16)), 0.0)
        h2parts.append(jnp.max(om, axis=0, keepdims=True))
    h2 = jnp.concatenate(h2parts, axis=1)
    hq_ref[e] = jnp.maximum(qenc, 0.0) * h2


def _head_body(hq_ref, wo1_ref, bo1_ref, wo2_ref, bo2_ref, out_ref, hid_s):
    j = pl.program_id(0)

    @pl.when(j == 0)
    def _init():
        hid_s[...] = jnp.maximum(
            _ntdot(hq_ref[...].astype(jnp.bfloat16), wo1_ref[...])
            + bo1_ref[...], 0.0)

    out_ref[...] = _ntdot(hid_s[...].astype(jnp.bfloat16), wo2_ref[...]) \
        + bo2_ref[...]


def kernel(question, image, K, W_lproj, b_lproj, W_e1, b_e1, W_e2, b_e2,
           Wg1, mr1, mt1, pr1, pt1, Wg2, mr2, mt2, pr2, pt2,
           W_o1, b_o1, W_o2, b_o2):
    del K
    b_lp = b_lproj.reshape(1, HID)
    b1 = b_e1.reshape(1, COMB)
    b2 = b_e2.reshape(1, COMB)
    bo1 = b_o1.reshape(1, OUTD)
    bo2 = b_o2.reshape(1, OUTD)
    p1 = jnp.stack([mr1, mt1, pr1, pt1])
    p2 = jnp.stack([mr2, mt2, pr2, pt2])

    f32 = jnp.float32
    full2 = lambda s: pl.BlockSpec(s, lambda b: (0, 0))

    adj, hq = pl.pallas_call(
        _fused_body,
        grid=(BB // PER_STEP,),
        in_specs=[
            full2((BB, EMB)),
            pl.BlockSpec((PER_STEP, KOBJ, FEAT), lambda b: (b, 0, 0)),
            full2((HID, EMB)),
            full2((1, HID)),
            full2((COMB, FEAT + HID)),
            full2((1, COMB)),
            full2((COMB, COMB)),
            full2((1, COMB)),
            pl.BlockSpec(memory_space=pltpu.SMEM),
            pl.BlockSpec(memory_space=pltpu.SMEM),
            pl.BlockSpec((NK, FEAT, 256), lambda b: (0, 0, 0)),
            pl.BlockSpec((NK, 2 * HID, 128), lambda b: (0, 0, 0)),
        ],
        out_specs=[
            pl.BlockSpec((PER_STEP, KOBJ, KOBJ), lambda b: (b, 0, 0)),
            pl.BlockSpec((PER_STEP, 1, HID), lambda b: (b, 0, 0)),
        ],
        out_shape=[
            jax.ShapeDtypeStruct((BB, KOBJ, KOBJ), f32),
            jax.ShapeDtypeStruct((BB, 1, HID), f32),
        ],
        scratch_shapes=[
            pltpu.VMEM((BB, HID), f32),
            pltpu.VMEM((BB, COMB), f32),
            pltpu.VMEM((NK, FEAT, 256), jnp.bfloat16),
            pltpu.VMEM((NK, 2 * HID, 128), jnp.bfloat16),
        ],
    )(question, image, W_lproj, b_lp, W_e1, b1, W_e2, b2, p1, p2, Wg1, Wg2)
    hq = hq.reshape(BB, HID)

    wo1b = W_o1.astype(jnp.bfloat16)
    wo2b = W_o2.astype(jnp.bfloat16)

    NT = 512
    logits = pl.pallas_call(
        _head_body,
        grid=(pl.cdiv(OUTD, NT),),
        in_specs=[
            pl.BlockSpec((BB, HID), lambda j: (0, 0)),
            pl.BlockSpec((OUTD, HID), lambda j: (0, 0)),
            pl.BlockSpec((1, OUTD), lambda j: (0, 0)),
            pl.BlockSpec((NT, OUTD), lambda j: (j, 0)),
            pl.BlockSpec((1, NT), lambda j: (0, j)),
        ],
        out_specs=pl.BlockSpec((BB, NT), lambda j: (0, j)),
        out_shape=jax.ShapeDtypeStruct((BB, OUTD), f32),
        scratch_shapes=[pltpu.VMEM((BB, OUTD), f32)],
    )(hq, wo1b, bo1, wo2b, bo2)

    return logits, adj


# speedup vs baseline: 1.1152x; 1.1152x over previous
import jax
import jax.numpy as jnp
import numpy as np
from jax.experimental import pallas as pl
from jax.experimental.pallas import tpu as pltpu

BB = 16
PER_STEP = 4
KOBJ = 100
EMB = 1024
FEAT = 2052
HID = 1024
OUTD = 3000
NK = 8
NS = 16
COMB = 512

_TWO_PI = np.float32(2.0 * np.pi)
_NEG = np.float32(-3.0e38)


def _f32dot(a, b):
    return jnp.dot(a, b, preferred_element_type=jnp.float32)


def _ntdot(a, b):
    return jax.lax.dot_general(a, b, (((1,), (1,)), ((), ())),
                               preferred_element_type=jnp.float32)


def _top16_mask(adj):
    cur = adj
    c = jnp.zeros((1, KOBJ), jnp.float32)
    v16 = jnp.zeros((1, KOBJ), jnp.float32)
    g = jnp.zeros((1, KOBJ), jnp.float32)
    rowmax = jnp.max(adj, axis=0, keepdims=True)
    for _ in range(NS):
        t = jnp.max(cur, axis=0, keepdims=True)
        hit = cur >= t
        nh = jnp.sum(hit.astype(jnp.float32), axis=0, keepdims=True)
        active = c < float(NS)
        v16 = jnp.where(active, t, v16)
        g = jnp.where(active, c, g)
        c = c + nh
        cur = jnp.where(hit, _NEG, cur)
    v16c = v16.T
    gc = g.T
    gt = adj > v16c
    eq = adj == v16c
    lidx = jax.lax.broadcasted_iota(jnp.int32, (KOBJ, KOBJ), 0)
    jidx = jax.lax.broadcasted_iota(jnp.int32, (KOBJ, KOBJ), 1)
    upper = (lidx < jidx).astype(jnp.float32)
    prefix = _f32dot(eq.astype(jnp.float32), upper)
    sel = jnp.logical_or(gt, jnp.logical_and(eq, prefix < (float(NS) - gc)))
    return sel, rowmax.T


def _kernel_weights(p_ref, rho, theta):
    ws = []
    wsum = jnp.zeros((KOBJ, KOBJ), jnp.float32)
    for m in range(NK):
        mr = p_ref[0, m]
        mt = p_ref[1, m]
        pr = p_ref[2, m]
        pt = p_ref[3, m]
        cr = -0.5 / (1e-14 + pr * pr)
        ct = -0.5 / (1e-14 + pt * pt)
        d = rho - mr
        wr = jnp.exp(d * d * cr)
        fa = jnp.abs(theta - mt)
        sa = jnp.abs(_TWO_PI - fa)
        mn = jnp.minimum(fa, sa)
        wt = jnp.exp(mn * mn * ct)
        w = wr * wt
        ws.append(w)
        wsum = wsum + w
    return ws, wsum


def _fused_body(qall_ref, img_ref, wlp_ref, blp_ref, w1_ref, b1_ref, w2_ref,
                b2_ref, p1_ref, p2_ref, wg1_ref, wg2_ref, adj_ref, hq_ref,
                qenc_s, qp_s, wg1_s, wg2_s):
    b = pl.program_id(0)

    @pl.when(b == 0)
    def _init():
        qe = _ntdot(qall_ref[...], wlp_ref[...]) + blp_ref[...]
        qenc_s[...] = qe
        qp_s[...] = _ntdot(qe, w1_ref[:, FEAT:])
        wg1_s[...] = wg1_ref[...].astype(jnp.bfloat16)
        wg2_s[...] = wg2_ref[...].astype(jnp.bfloat16)

    for e in range(PER_STEP):
        _one_example(b * PER_STEP + e, e, img_ref, w1_ref, b1_ref, w2_ref,
                     b2_ref, p1_ref, p2_ref, adj_ref, hq_ref,
                     qenc_s, qp_s, wg1_s, wg2_s)


def _one_example(bex, e, img_ref, w1_ref, b1_ref, w2_ref, b2_ref,
                 p1_ref, p2_ref, adj_ref, hq_ref, qenc_s, qp_s, wg1_s, wg2_s):
    qenc = qenc_s[pl.ds(bex, 1), :]
    qpart = qp_s[pl.ds(bex, 1), :]
    img = img_ref[e]
    h = jnp.maximum(
        _ntdot(img, w1_ref[:, :FEAT]) + qpart + b1_ref[...], 0.0)
    h = jnp.maximum(_ntdot(h, w2_ref[...]) + b2_ref[...], 0.0)
    adj = _ntdot(h, h)
    adj_ref[e] = adj

    bb = img[:, FEAT - 4:]
    cx = 0.5 * (bb[:, 0:1] + bb[:, 2:3])
    cy = 0.5 * (bb[:, 1:2] + bb[:, 3:4])
    dx = cx - cx.T
    dy = cy - cy.T
    rho = jnp.sqrt(dx * dx + dy * dy)
    theta = jnp.arctan2(dx, dy)

    sel, rowmaxc = _top16_mask(adj)
    ex = jnp.where(sel, jnp.exp(adj - rowmaxc), 0.0)
    sm = ex * (1.0 / jnp.sum(ex, axis=1, keepdims=True))
    self32 = sel.astype(jnp.float32)
    bf16 = jnp.bfloat16
    imgb = img.astype(bf16)

    ws1, wsum1 = _kernel_weights(p1_ref, rho, theta)
    s1 = sm * (1.0 / wsum1)
    h1parts = []
    for m in range(NK):
        y = _f32dot(imgb, wg1_s[m])
        a1m = jnp.where(sel, ws1[m] * s1, 0.0)
        h1parts.append(
            jnp.maximum(_f32dot(a1m.astype(bf16), y.astype(bf16)), 0.0))
    h1 = jnp.concatenate(h1parts, axis=1)

    ws2, wsum2 = _kernel_weights(p2_ref, rho, theta)
    s2 = self32 * (1.0 / wsum2)
    h1b = h1.astype(bf16)
    h2parts = []
    for m in range(NK):
        y2 = _f32dot(h1b, wg2_s[m])
        a2m = jnp.where(sel, ws2[m] * s2, 0.0)
        om = jnp.maximum(_f32dot(a2m.astype(bf16), y2.astype(bf16)), 0.0)
        h2parts.append(jnp.max(om, axis=0, keepdims=True))
    h2 = jnp.concatenate(h2parts, axis=1)
    hq_ref[e] = jnp.maximum(qenc, 0.0) * h2


def _head_body(hq_ref, wo1_ref, bo1_ref, wo2_ref, bo2_ref, out_ref, hid_s):
    j = pl.program_id(0)

    @pl.when(j == 0)
    def _init():
        hid_s[...] = jnp.maximum(
            _ntdot(hq_ref[...], wo1_ref[...]) + bo1_ref[...], 0.0)

    out_ref[...] = _ntdot(hid_s[...], wo2_ref[...]) + bo2_ref[...]


def kernel(question, image, K, W_lproj, b_lproj, W_e1, b_e1, W_e2, b_e2,
           Wg1, mr1, mt1, pr1, pt1, Wg2, mr2, mt2, pr2, pt2,
           W_o1, b_o1, W_o2, b_o2):
    del K
    b_lp = b_lproj.reshape(1, HID)
    b1 = b_e1.reshape(1, COMB)
    b2 = b_e2.reshape(1, COMB)
    bo1 = b_o1.reshape(1, OUTD)
    bo2 = b_o2.reshape(1, OUTD)
    p1 = jnp.stack([mr1, mt1, pr1, pt1])
    p2 = jnp.stack([mr2, mt2, pr2, pt2])

    f32 = jnp.float32
    full2 = lambda s: pl.BlockSpec(s, lambda b: (0, 0))

    adj, hq = pl.pallas_call(
        _fused_body,
        grid=(BB // PER_STEP,),
        in_specs=[
            full2((BB, EMB)),
            pl.BlockSpec((PER_STEP, KOBJ, FEAT), lambda b: (b, 0, 0)),
            full2((HID, EMB)),
            full2((1, HID)),
            full2((COMB, FEAT + HID)),
            full2((1, COMB)),
            full2((COMB, COMB)),
            full2((1, COMB)),
            pl.BlockSpec(memory_space=pltpu.SMEM),
            pl.BlockSpec(memory_space=pltpu.SMEM),
            pl.BlockSpec((NK, FEAT, 256), lambda b: (0, 0, 0)),
            pl.BlockSpec((NK, 2 * HID, 128), lambda b: (0, 0, 0)),
        ],
        out_specs=[
            pl.BlockSpec((PER_STEP, KOBJ, KOBJ), lambda b: (b, 0, 0)),
            pl.BlockSpec((PER_STEP, 1, HID), lambda b: (b, 0, 0)),
        ],
        out_shape=[
            jax.ShapeDtypeStruct((BB, KOBJ, KOBJ), f32),
            jax.ShapeDtypeStruct((BB, 1, HID), f32),
        ],
        scratch_shapes=[
            pltpu.VMEM((BB, HID), f32),
            pltpu.VMEM((BB, COMB), f32),
            pltpu.VMEM((NK, FEAT, 256), jnp.bfloat16),
            pltpu.VMEM((NK, 2 * HID, 128), jnp.bfloat16),
        ],
    )(question, image, W_lproj, b_lp, W_e1, b1, W_e2, b2, p1, p2, Wg1, Wg2)
    hq = hq.reshape(BB, HID)

    NT = 512
    logits = pl.pallas_call(
        _head_body,
        grid=(pl.cdiv(OUTD, NT),),
        in_specs=[
            pl.BlockSpec((BB, HID), lambda j: (0, 0)),
            pl.BlockSpec((OUTD, HID), lambda j: (0, 0)),
            pl.BlockSpec((1, OUTD), lambda j: (0, 0)),
            pl.BlockSpec((NT, OUTD), lambda j: (j, 0)),
            pl.BlockSpec((1, NT), lambda j: (0, j)),
        ],
        out_specs=pl.BlockSpec((BB, NT), lambda j: (0, j)),
        out_shape=jax.ShapeDtypeStruct((BB, OUTD), f32),
        scratch_shapes=[pltpu.VMEM((BB, OUTD), f32)],
    )(hq, W_o1, bo1, W_o2, bo2)

    return logits, adj
